# baseline (speedup 1.0000x reference)
import functools

import jax
import jax.numpy as jnp
from jax import lax
from jax.experimental import pallas as pl
from jax.experimental.pallas import tpu as pltpu
from jax.experimental.pallas import tpu_sc as plsc

NUM_CHAMPIONS = 1000
NUM_PATCHES = 100
EMBED_DIM = 64
MODULATION_SCALE = 0.2
BATCH = 16384
NUM_POS = 20

NC = 2
NS = 16
L = 16
NW = NC * NS

BATCH_PER_W = BATCH // NW
HALF = BATCH_PER_W // 2
DSTRIPS = EMBED_DIM // 8
BTILES = BATCH // 128
OUT_ROWS = NUM_POS * DSTRIPS * BTILES

_TBL_ROWS = NUM_PATCHES * NUM_CHAMPIONS * EMBED_DIM // 128
_TBL_BLK = NUM_CHAMPIONS * EMBED_DIM // 128


def _table_body(mt_ref, base_t_ref, out_ref):
    bt = jnp.swapaxes(base_t_ref[...], 0, 1)
    nb = _HROWS // NUM_CHAMPIONS
    for blk in range(nb):
        sl = pl.ds(blk * NUM_CHAMPIONS, NUM_CHAMPIONS)
        sl2 = pl.ds(_HROWS + blk * NUM_CHAMPIONS, NUM_CHAMPIONS)
        t1 = jnp.swapaxes(mt_ref[:, sl], 0, 1)
        t2 = jnp.swapaxes(mt_ref[:, sl2], 0, 1)
        m1 = bt * (1.0 + MODULATION_SCALE * jnp.tanh(t1))
        m2 = bt * (1.0 + MODULATION_SCALE * jnp.tanh(t2))
        out_ref[sl, :] = jnp.concatenate([m1, m2], axis=1)


_HROWS = NUM_PATCHES * NUM_CHAMPIONS // 2


def _build_table(patch_modulation, champion_base):
    mod_t = patch_modulation.T
    base_t = champion_base.T
    table = pl.pallas_call(
        _table_body,
        grid=(1,),
        in_specs=[
            pl.BlockSpec((EMBED_DIM, 2 * _HROWS), lambda i: (0, 0)),
            pl.BlockSpec((EMBED_DIM, NUM_CHAMPIONS), lambda i: (0, 0)),
        ],
        out_specs=pl.BlockSpec((_HROWS, 128), lambda i: (0, 0)),
        out_shape=jax.ShapeDtypeStruct((_HROWS, 128), jnp.float32),
    )(mod_t, base_t)
    return table.reshape(NUM_PATCHES * NUM_CHAMPIONS, EMBED_DIM)


def _gather_body(table_hbm, cid_hbm, patch_hbm, out_hbm,
                 patch_v, cmb_a, cmb_b, gbuf_a, gbuf_b, tbuf_a, tbuf_b,
                 gsem_a, gsem_b, wsem_a, wsem_b):
    wid = lax.axis_index("s") * NC + lax.axis_index("c")
    b0 = wid * BATCH_PER_W
    bt0 = wid * (BATCH_PER_W // 128)

    pltpu.sync_copy(patch_hbm.at[pl.ds(b0, BATCH_PER_W)], patch_v)

    iota = lax.iota(jnp.int32, L)

    def start_gathers(p, h, cmb, gbuf, sem):
        pltpu.sync_copy(cid_hbm.at[p, pl.ds(bt0 + 2 * h, 2)], cmb)
        for k in range(2):
            for g in range(128 // L):
                pat = patch_v[pl.ds(h * HALF + k * 128 + g * L, L)]
                jj = pat * NUM_CHAMPIONS + cmb[k, pl.ds(g * L, L)]
                half = lax.div(jj, _HROWS)
                cmb[k, pl.ds(g * L, L)] = 2 * jj - (2 * _HROWS - 1) * half
        for k in range(2):
            pltpu.async_copy(table_hbm.at[cmb.at[k]],
                             gbuf.at[pl.ds(k * 128, 128)], sem)

    def wait_gathers(cmb, gbuf, sem):
        for k in range(2):
            pltpu.make_async_copy(table_hbm.at[cmb.at[k]],
                                  gbuf.at[pl.ds(k * 128, 128)], sem).wait()

    lane_dlow = lax.rem(iota, 8)
    lane_tvoff = lax.div(iota, 8) * 2
    izero = iota * 0

    tv_vecs = [[lane_tvoff + (4 * v + k) for v in range(EMBED_DIM // L)]
               for k in range(2)]

    def transpose(gbuf, tbuf):
        for k in range(2):
            def b_body(b2, carry, k=k):
                blow_vec = izero + b2
                b = k * 128 + b2
                for v in range(EMBED_DIM // L):
                    x = gbuf[b, pl.ds(v * L, L)]
                    plsc.store_scatter(
                        tbuf, [tv_vecs[k][v], lane_dlow, blow_vec], x)
                return carry

            lax.fori_loop(0, 128, b_body, 0, unroll=4)

    def start_writes(p, h, tbuf, sem):
        for tv in range(16):
            t0 = (p * DSTRIPS + tv // 2) * BTILES + bt0 + 2 * h + tv % 2
            pltpu.async_copy(tbuf.at[tv, pl.ds(0, 8), pl.ds(0, 128)],
                             out_hbm.at[t0], sem)

    def wait_writes(tbuf, sem):
        for tv in range(16):
            pltpu.make_async_copy(tbuf.at[tv, pl.ds(0, 8), pl.ds(0, 128)],
                                  out_hbm.at[tv], sem).wait()

    start_gathers(0, 0, cmb_a, gbuf_a, gsem_a)

    def body(p, carry):
        start_gathers(p, 1, cmb_b, gbuf_b, gsem_b)
        wait_gathers(cmb_a, gbuf_a, gsem_a)

        @pl.when(p > 0)
        def _():
            wait_writes(tbuf_a, wsem_a)

        transpose(gbuf_a, tbuf_a)
        start_writes(p, 0, tbuf_a, wsem_a)

        @pl.when(p + 1 < NUM_POS)
        def _():
            start_gathers(p + 1, 0, cmb_a, gbuf_a, gsem_a)

        wait_gathers(cmb_b, gbuf_b, gsem_b)

        @pl.when(p > 0)
        def _():
            wait_writes(tbuf_b, wsem_b)

        transpose(gbuf_b, tbuf_b)
        start_writes(p, 1, tbuf_b, wsem_b)
        return carry

    lax.fori_loop(0, NUM_POS, body, 0)
    wait_writes(tbuf_a, wsem_a)
    wait_writes(tbuf_b, wsem_b)


@functools.lru_cache(maxsize=1)
def _make_sc_gather():
    return pl.kernel(
        _gather_body,
        out_type=jax.ShapeDtypeStruct((OUT_ROWS, 8, 128), jnp.float32),
        mesh=plsc.VectorSubcoreMesh(core_axis_name="c", subcore_axis_name="s",
                                    num_cores=NC, num_subcores=NS),
        compiler_params=pltpu.CompilerParams(use_tc_tiling_on_sc=False,
                                             needs_layout_passes=False),
        scratch_types=[
            pltpu.VMEM((BATCH_PER_W,), jnp.int32),
            pltpu.VMEM((2, 128), jnp.int32),
            pltpu.VMEM((2, 128), jnp.int32),
            pltpu.VMEM((HALF, EMBED_DIM), jnp.float32),
            pltpu.VMEM((HALF, EMBED_DIM), jnp.float32),
            pltpu.VMEM((16, 9, 131), jnp.float32),
            pltpu.VMEM((16, 9, 131), jnp.float32),
            pltpu.SemaphoreType.DMA,
            pltpu.SemaphoreType.DMA,
            pltpu.SemaphoreType.DMA,
            pltpu.SemaphoreType.DMA,
        ],
    )


def kernel(champion_ids, patch_ids, champion_base, patch_modulation):
    table = _build_table(patch_modulation, champion_base)
    cid_t = champion_ids.astype(jnp.int32).T.reshape(NUM_POS, BTILES, 128)
    out = _make_sc_gather()(table, cid_t, patch_ids.astype(jnp.int32))
    o5 = out.reshape(NUM_POS, DSTRIPS, BTILES, 8, 128)
    return o5.transpose(2, 4, 0, 1, 3).reshape(BATCH, NUM_POS, EMBED_DIM)


# speedup vs baseline: 1.2213x; 1.2213x over previous
import functools

import jax
import jax.numpy as jnp
from jax import lax
from jax.experimental import pallas as pl
from jax.experimental.pallas import tpu as pltpu
from jax.experimental.pallas import tpu_sc as plsc

NUM_CHAMPIONS = 1000
NUM_PATCHES = 100
EMBED_DIM = 64
MODULATION_SCALE = 0.2
BATCH = 16384
NUM_POS = 20

NC = 2
NS = 16
L = 16
NW = NC * NS

BATCH_PER_W = BATCH // NW
HALF = BATCH_PER_W // 2
DSTRIPS = EMBED_DIM // 8
BTILES = BATCH // 128
OUT_ROWS = NUM_POS * DSTRIPS * BTILES

_TBL_ROWS = NUM_PATCHES * NUM_CHAMPIONS * EMBED_DIM // 128
_TBL_BLK = NUM_CHAMPIONS * EMBED_DIM // 128


def _table_body(mod_ref, base_ref, out_ref):
    b = base_ref[...]
    b2 = jnp.concatenate([b, b], axis=0)
    out_ref[...] = b2 * (1.0 + MODULATION_SCALE * jnp.tanh(mod_ref[...]))


def _build_table(patch_modulation, champion_base):
    mod2 = patch_modulation.reshape(_TBL_ROWS, 128)
    base2 = champion_base.reshape(_TBL_BLK, 128)
    table = pl.pallas_call(
        _table_body,
        grid=(NUM_PATCHES // 2,),
        in_specs=[
            pl.BlockSpec((2 * _TBL_BLK, 128), lambda i: (i, 0)),
            pl.BlockSpec((_TBL_BLK, 128), lambda i: (0, 0)),
        ],
        out_specs=pl.BlockSpec((2 * _TBL_BLK, 128), lambda i: (i, 0)),
        out_shape=jax.ShapeDtypeStruct((_TBL_ROWS, 128), jnp.float32),
    )(mod2, base2)
    return table.reshape(NUM_PATCHES * NUM_CHAMPIONS, EMBED_DIM)


def _gather_body(table_hbm, cid_hbm, patch_hbm, out_hbm,
                 patch_v, cid_v, cmb_a, cmb_b, gbuf_a, gbuf_b, tbuf_a, tbuf_b,
                 gsem_a, gsem_b, wsem_a, wsem_b):
    wid = lax.axis_index("s") * NC + lax.axis_index("c")
    b0 = wid * BATCH_PER_W
    bt0 = wid * (BATCH_PER_W // 128)

    pltpu.sync_copy(patch_hbm.at[pl.ds(b0, BATCH_PER_W)], patch_v)
    pltpu.sync_copy(cid_hbm.at[:, pl.ds(bt0, 4)], cid_v)

    iota = lax.iota(jnp.int32, L)

    def start_gathers(p, h, cmb, gbuf, sem):
        for k in range(2):
            for g in range(128 // L):
                pat = patch_v[pl.ds(h * HALF + k * 128 + g * L, L)]
                cid = cid_v[p, 2 * h + k, pl.ds(g * L, L)]
                cmb[k, pl.ds(g * L, L)] = pat * NUM_CHAMPIONS + cid
        for k in range(2):
            pltpu.async_copy(table_hbm.at[cmb.at[k]],
                             gbuf.at[pl.ds(k * 128, 128)], sem)

    def wait_gathers(cmb, gbuf, sem):
        for k in range(2):
            pltpu.make_async_copy(table_hbm.at[cmb.at[k]],
                                  gbuf.at[pl.ds(k * 128, 128)], sem).wait()

    lane_dlow = lax.rem(iota, 8)
    lane_tvoff = lax.div(iota, 8) * 2
    izero = iota * 0

    tv_vecs = [[lane_tvoff + (4 * v + k) for v in range(EMBED_DIM // L)]
               for k in range(2)]

    def transpose(gbuf, tbuf):
        for k in range(2):
            def b_body(b2, carry, k=k):
                blow_vec = izero + b2
                b = k * 128 + b2
                for v in range(EMBED_DIM // L):
                    x = gbuf[b, pl.ds(v * L, L)]
                    plsc.store_scatter(
                        tbuf, [tv_vecs[k][v], lane_dlow, blow_vec], x)
                return carry

            lax.fori_loop(0, 128, b_body, 0, unroll=4)

    def start_writes(p, h, tbuf, sem):
        for tv in range(16):
            t0 = (p * DSTRIPS + tv // 2) * BTILES + bt0 + 2 * h + tv % 2
            pltpu.async_copy(tbuf.at[tv, pl.ds(0, 8), pl.ds(0, 128)],
                             out_hbm.at[t0], sem)

    def wait_writes(tbuf, sem):
        for tv in range(16):
            pltpu.make_async_copy(tbuf.at[tv, pl.ds(0, 8), pl.ds(0, 128)],
                                  out_hbm.at[tv], sem).wait()

    start_gathers(0, 0, cmb_a, gbuf_a, gsem_a)

    def body(p, carry):
        start_gathers(p, 1, cmb_b, gbuf_b, gsem_b)
        wait_gathers(cmb_a, gbuf_a, gsem_a)

        @pl.when(p > 0)
        def _():
            wait_writes(tbuf_a, wsem_a)

        transpose(gbuf_a, tbuf_a)
        start_writes(p, 0, tbuf_a, wsem_a)

        @pl.when(p + 1 < NUM_POS)
        def _():
            start_gathers(p + 1, 0, cmb_a, gbuf_a, gsem_a)

        wait_gathers(cmb_b, gbuf_b, gsem_b)

        @pl.when(p > 0)
        def _():
            wait_writes(tbuf_b, wsem_b)

        transpose(gbuf_b, tbuf_b)
        start_writes(p, 1, tbuf_b, wsem_b)
        return carry

    lax.fori_loop(0, NUM_POS, body, 0)
    wait_writes(tbuf_a, wsem_a)
    wait_writes(tbuf_b, wsem_b)


@functools.lru_cache(maxsize=1)
def _make_sc_gather():
    return pl.kernel(
        _gather_body,
        out_type=jax.ShapeDtypeStruct((OUT_ROWS, 8, 128), jnp.float32),
        mesh=plsc.VectorSubcoreMesh(core_axis_name="c", subcore_axis_name="s",
                                    num_cores=NC, num_subcores=NS),
        compiler_params=pltpu.CompilerParams(use_tc_tiling_on_sc=False,
                                             needs_layout_passes=False),
        scratch_types=[
            pltpu.VMEM((BATCH_PER_W,), jnp.int32),
            pltpu.VMEM((NUM_POS, 4, 128), jnp.int32),
            pltpu.VMEM((2, 128), jnp.int32),
            pltpu.VMEM((2, 128), jnp.int32),
            pltpu.VMEM((HALF, EMBED_DIM), jnp.float32),
            pltpu.VMEM((HALF, EMBED_DIM), jnp.float32),
            pltpu.VMEM((16, 9, 131), jnp.float32),
            pltpu.VMEM((16, 9, 131), jnp.float32),
            pltpu.SemaphoreType.DMA,
            pltpu.SemaphoreType.DMA,
            pltpu.SemaphoreType.DMA,
            pltpu.SemaphoreType.DMA,
        ],
    )


def kernel(champion_ids, patch_ids, champion_base, patch_modulation):
    table = _build_table(patch_modulation, champion_base)
    cid_t = champion_ids.astype(jnp.int32).T.reshape(NUM_POS, BTILES, 128)
    out = _make_sc_gather()(table, cid_t, patch_ids.astype(jnp.int32))
    o5 = out.reshape(NUM_POS, DSTRIPS, BTILES, 8, 128)
    return o5.transpose(2, 4, 0, 1, 3).reshape(BATCH, NUM_POS, EMBED_DIM)
